# baseline (speedup 1.0000x reference)
import functools

import jax
import jax.numpy as jnp
from jax import lax
from jax.experimental import pallas as pl
from jax.experimental.pallas import tpu as pltpu
from jax.experimental.pallas import tpu_sc as plsc

_L = 16
_NW = 32


def _make_emb_kernel(BATCH, HIST, D, V2):
    b_per_w = BATCH // _NW
    nbb = b_per_w // 128
    NBLK = nbb * HIST
    i_per_w = b_per_w * HIST
    NV = 128 // _L

    mesh = plsc.VectorSubcoreMesh(core_axis_name="c", subcore_axis_name="s")
    nc = mesh.num_cores

    scratch = [
        pltpu.VMEM((i_per_w,), jnp.int32),
        pltpu.VMEM((128,), jnp.int32),
        pltpu.VMEM((128,), jnp.int32),
        pltpu.VMEM((128, 2 * D), jnp.float32),
        pltpu.VMEM((128, 2 * D), jnp.float32),
        pltpu.VMEM((D, 128), jnp.float32),
        pltpu.VMEM((D, 128), jnp.float32),
        pltpu.SemaphoreType.DMA,
        pltpu.SemaphoreType.DMA,
        pltpu.SemaphoreType.DMA,
        pltpu.SemaphoreType.DMA,
    ]

    @functools.partial(
        pl.kernel,
        out_type=jax.ShapeDtypeStruct((HIST, D, BATCH), jnp.float32),
        mesh=mesh,
        scratch_types=scratch,
        compiler_params=pltpu.CompilerParams(
            needs_layout_passes=False, disable_bounds_checks=True
        ),
    )
    def emb(idx_hbm, tab_hbm, out_hbm, idx_v, gi0, gi1, rw0, rw1, ob0, ob1,
            gs0, gs1, ss0, ss1):
        gi = (gi0, gi1)
        rw = (rw0, rw1)
        ob = (ob0, ob1)
        gsem = (gs0, gs1)
        ssem = (ss0, ss1)
        wid = lax.axis_index("s") * nc + lax.axis_index("c")
        ibase = wid * i_per_w
        pltpu.sync_copy(idx_hbm.at[pl.ds(ibase, i_per_w)], idx_v)

        lanes = lax.iota(jnp.int32, _L)
        jofs = [(lanes + v * _L) * HIST for v in range(NV)]
        rowv = [lanes + v * _L for v in range(NV)]

        def blk_h_bb(k):
            return lax.rem(k, HIST), lax.div(k, HIST)

        def prep_gather(k, p):
            h, bb = blk_h_bb(k)
            base = bb * (128 * HIST) + h
            for v in range(NV):
                raw = plsc.load_gather(idx_v, [jofs[v] + base])
                gi[p][pl.ds(v * _L, _L)] = lax.shift_right_logical(raw, 1)
            return pltpu.async_copy(tab_hbm.at[gi[p]], rw[p], gsem[p])

        rowbase = [(lanes + v * _L) * (2 * D) for v in range(NV)]
        zerov = lanes * 0

        def transpose_block(k, p):
            h, bb = blk_h_bb(k)
            base = bb * (128 * HIST) + h
            raws = [plsc.load_gather(idx_v, [jofs[v] + base])
                    for v in range(NV)]
            pcols = [(r & 1) * D for r in raws]

            @plsc.parallel_loop(0, D, unroll=16)
            def _(d):
                for v in range(NV):
                    val = plsc.load_gather(rw[p], [rowv[v], pcols[v] + d])
                    ob[p][d, pl.ds(v * _L, _L)] = val

        def writeback(k, p):
            h, bb = blk_h_bb(k)
            bg = (wid * nbb + bb) * 128
            return pltpu.async_copy(
                ob[p], out_hbm.at[h, :, pl.ds(bg, 128)], ssem[p]
            )

        def wb_wait(p):
            pltpu.make_async_copy(
                ob[p], out_hbm.at[0, :, pl.ds(0, 128)], ssem[p]
            ).wait()

        g0 = prep_gather(0, 0)
        g1 = prep_gather(1, 1)
        g0.wait()
        transpose_block(0, 0)
        writeback(0, 0)

        def lap(L_, carry):
            k = 2 * L_ - 1
            prep_gather(k + 1, 0)
            pltpu.make_async_copy(tab_hbm.at[gi[1]], rw[1], gsem[1]).wait()

            @pl.when(L_ > 1)
            def _():
                wb_wait(1)

            transpose_block(k, 1)
            writeback(k, 1)

            prep_gather(k + 2, 1)
            pltpu.make_async_copy(tab_hbm.at[gi[0]], rw[0], gsem[0]).wait()
            wb_wait(0)
            transpose_block(k + 1, 0)
            writeback(k + 1, 0)
            return carry

        lax.fori_loop(1, (NBLK - 2) // 2 + 1, lap, 0)

        k = NBLK - 1
        pltpu.make_async_copy(tab_hbm.at[gi[1]], rw[1], gsem[1]).wait()
        wb_wait(1)
        transpose_block(k, 1)
        writeback(k, 1)

        wb_wait(0)
        wb_wait(1)

    return emb


def kernel(text_input_ids, embedding_table):
    BATCH, HIST = text_input_ids.shape
    V, D = embedding_table.shape
    idx = text_input_ids.reshape(-1).astype(jnp.int32)
    tab2 = embedding_table.reshape(V // 2, 2 * D)
    emb = _make_emb_kernel(BATCH, HIST, D, V // 2)
    out = emb(idx, tab2)
    return jnp.transpose(out, (2, 0, 1))


# speedup vs baseline: 1.0039x; 1.0039x over previous
import functools

import jax
import jax.numpy as jnp
from jax import lax
from jax.experimental import pallas as pl
from jax.experimental.pallas import tpu as pltpu
from jax.experimental.pallas import tpu_sc as plsc

_L = 16
_NW = 32


def _make_emb_kernel(BATCH, HIST, D):
    b_per_w = BATCH // _NW
    nbb = b_per_w // 128
    HP = HIST // 2
    NBLK = nbb * HP
    i_per_w = b_per_w * HIST
    NV = 128 // _L

    mesh = plsc.VectorSubcoreMesh(core_axis_name="c", subcore_axis_name="s")
    nc = mesh.num_cores

    scratch = [
        pltpu.VMEM((i_per_w,), jnp.int32),
        pltpu.VMEM((256,), jnp.int32),
        pltpu.VMEM((256,), jnp.int32),
        pltpu.VMEM((256, 2 * D), jnp.float32),
        pltpu.VMEM((256, 2 * D), jnp.float32),
        pltpu.VMEM((2, D, 128), jnp.float32),
        pltpu.VMEM((2, D, 128), jnp.float32),
        pltpu.SemaphoreType.DMA,
        pltpu.SemaphoreType.DMA,
        pltpu.SemaphoreType.DMA,
        pltpu.SemaphoreType.DMA,
    ]

    @functools.partial(
        pl.kernel,
        out_type=jax.ShapeDtypeStruct((HIST, D, BATCH), jnp.float32),
        mesh=mesh,
        scratch_types=scratch,
        compiler_params=pltpu.CompilerParams(
            needs_layout_passes=False, disable_bounds_checks=True
        ),
    )
    def emb(idx_hbm, tab_hbm, out_hbm, idx_v, gi0, gi1, rw0, rw1, ob0, ob1,
            gs0, gs1, ss0, ss1):
        gi = (gi0, gi1)
        rw = (rw0, rw1)
        ob = (ob0, ob1)
        gsem = (gs0, gs1)
        ssem = (ss0, ss1)
        wid = lax.axis_index("s") * nc + lax.axis_index("c")
        ibase = wid * i_per_w
        pltpu.sync_copy(idx_hbm.at[pl.ds(ibase, i_per_w)], idx_v)

        lanes = lax.iota(jnp.int32, _L)
        jofs = [(lanes + v * _L) * HIST for v in range(NV)]
        rowv = [lanes + v * _L for v in range(NV)]
        rowv2 = [lanes + v * _L + 128 for v in range(NV)]

        def blk_h_bb(k):
            return 2 * lax.rem(k, HP), lax.div(k, HP)

        def load_raws(k):
            h, bb = blk_h_bb(k)
            base = bb * (128 * HIST) + h
            rawsA = [plsc.load_gather(idx_v, [jofs[v] + base]) for v in range(NV)]
            rawsB = [plsc.load_gather(idx_v, [jofs[v] + base + 1]) for v in range(NV)]
            return rawsA, rawsB

        def prep_gather(k, p):
            rawsA, rawsB = load_raws(k)
            for v in range(NV):
                gi[p][pl.ds(v * _L, _L)] = lax.shift_right_logical(rawsA[v], 1)
                gi[p][pl.ds(128 + v * _L, _L)] = lax.shift_right_logical(rawsB[v], 1)
            return pltpu.async_copy(tab_hbm.at[gi[p]], rw[p], gsem[p])

        def transpose_block(k, p):
            rawsA, rawsB = load_raws(k)
            pcA = [(r & 1) * D for r in rawsA]
            pcB = [(r & 1) * D for r in rawsB]

            @plsc.parallel_loop(0, D, unroll=8)
            def _(d):
                for v in range(NV):
                    ob[p][0, d, pl.ds(v * _L, _L)] = plsc.load_gather(
                        rw[p], [rowv[v], pcA[v] + d]
                    )
                    ob[p][1, d, pl.ds(v * _L, _L)] = plsc.load_gather(
                        rw[p], [rowv2[v], pcB[v] + d]
                    )

        def writeback(k, p):
            h, bb = blk_h_bb(k)
            bg = (wid * nbb + bb) * 128
            return pltpu.async_copy(
                ob[p], out_hbm.at[pl.ds(h, 2), :, pl.ds(bg, 128)], ssem[p]
            )

        def wb_wait(p):
            pltpu.make_async_copy(
                ob[p], out_hbm.at[pl.ds(0, 2), :, pl.ds(0, 128)], ssem[p]
            ).wait()

        g0 = prep_gather(0, 0)
        g1 = prep_gather(1, 1)
        g0.wait()
        transpose_block(0, 0)
        writeback(0, 0)

        def lap(L_, carry):
            k = 2 * L_ - 1
            prep_gather(k + 1, 0)
            pltpu.make_async_copy(tab_hbm.at[gi[1]], rw[1], gsem[1]).wait()

            @pl.when(L_ > 1)
            def _():
                wb_wait(1)

            transpose_block(k, 1)
            writeback(k, 1)

            prep_gather(k + 2, 1)
            pltpu.make_async_copy(tab_hbm.at[gi[0]], rw[0], gsem[0]).wait()
            wb_wait(0)
            transpose_block(k + 1, 0)
            writeback(k + 1, 0)
            return carry

        lax.fori_loop(1, (NBLK - 2) // 2 + 1, lap, 0)

        k = NBLK - 1
        pltpu.make_async_copy(tab_hbm.at[gi[1]], rw[1], gsem[1]).wait()
        wb_wait(1)
        transpose_block(k, 1)
        writeback(k, 1)

        wb_wait(0)
        wb_wait(1)

    return emb


def kernel(text_input_ids, embedding_table):
    BATCH, HIST = text_input_ids.shape
    V, D = embedding_table.shape
    idx = text_input_ids.reshape(-1).astype(jnp.int32)
    tab2 = embedding_table.reshape(V // 2, 2 * D)
    emb = _make_emb_kernel(BATCH, HIST, D)
    out = emb(idx, tab2)
    return jnp.transpose(out, (2, 0, 1))
